# Initial kernel scaffold; baseline (speedup 1.0000x reference)
import jax
import jax.numpy as jnp
from jax.experimental import pallas as pl


def kernel(edge_scores, edge_index, edge_batch, node_global_ids, start_node_locals, current_tail, prev_tail, step_counts):
    raise NotImplementedError("write your pallas kernel here")


# speedup vs baseline: 252.8553x; 252.8553x over previous
import jax
import jax.numpy as jnp
from jax import lax
from jax.experimental import pallas as pl
from jax.experimental.pallas import tpu as pltpu
from jax.experimental.pallas import tpu_sc as plsc

_N_NODES = 50000
_N_EDGES = 1600000
_N_GRAPHS = 16
_N_STARTS = 64
_LANES = 16
_NUM_TILES = 32
_EDGES_PER_TILE = _N_EDGES // _NUM_TILES
_CHUNK = 2000
_CHUNKS_PER_TILE = _EDGES_PER_TILE // _CHUNK
_VECS_PER_CHUNK = _CHUNK // _LANES
_START_BIT = 1 << 30
_GID_MASK = _START_BIT - 1


def _body(scores, heads, tails, batch, gid, starts, ct, pt, stc, out,
          table_v, starts_v, ct_v, pt_v, st_v,
          h_v, t_v, b_v, sc_v, o_v):
    wid = lax.axis_index("s") * 2 + lax.axis_index("c")
    tile_base = wid * _EDGES_PER_TILE

    pltpu.sync_copy(gid, table_v)
    pltpu.sync_copy(starts, starts_v)
    pltpu.sync_copy(ct, ct_v)
    pltpu.sync_copy(pt, pt_v)
    pltpu.sync_copy(stc, st_v)

    for j in range(_N_STARTS // _LANES):
        sidx = starts_v[pl.ds(j * _LANES, _LANES)]
        cur = plsc.load_gather(table_v, [sidx])
        plsc.store_scatter(table_v, [sidx], cur | _START_BIT)

    def chunk_body(c, carry):
        base = tile_base + c * _CHUNK
        pltpu.sync_copy(scores.at[pl.ds(base, _CHUNK)], sc_v)
        pltpu.sync_copy(heads.at[pl.ds(base, _CHUNK)], h_v)
        pltpu.sync_copy(tails.at[pl.ds(base, _CHUNK)], t_v)
        pltpu.sync_copy(batch.at[pl.ds(base, _CHUNK)], b_v)

        def vec_body(i, c2):
            off = i * _LANES
            h = h_v[pl.ds(off, _LANES)]
            t = t_v[pl.ds(off, _LANES)]
            b = b_v[pl.ds(off, _LANES)]
            sc = sc_v[pl.ds(off, _LANES)]
            hv = plsc.load_gather(table_v, [h])
            tv = plsc.load_gather(table_v, [t])
            ctv = plsc.load_gather(ct_v, [b])
            ptv = plsc.load_gather(pt_v, [b])
            sv = plsc.load_gather(st_v, [b])
            hm = hv & _GID_MASK
            tm = tv & _GID_MASK
            hc = hm == ctv
            tc = tm == ctv
            hp = hm == ptv
            tp = tm == ptv
            valid = (hc | tc) & jnp.logical_not((hc & tp) | (tc & hp))
            is_start = ((hv | tv) & _START_BIT) != 0
            mask = jnp.where(sv == 0, is_start, valid)
            o_v[pl.ds(off, _LANES)] = jnp.where(mask, sc, jnp.float32(-1e9))
            return c2

        lax.fori_loop(0, _VECS_PER_CHUNK, vec_body, 0)
        pltpu.sync_copy(o_v, out.at[pl.ds(base, _CHUNK)])
        return carry

    lax.fori_loop(0, _CHUNKS_PER_TILE, chunk_body, 0)


def kernel(edge_scores, edge_index, edge_batch, node_global_ids,
           start_node_locals, current_tail, prev_tail, step_counts):
    mesh = plsc.VectorSubcoreMesh(core_axis_name="c", subcore_axis_name="s")
    run = pl.kernel(
        _body,
        mesh=mesh,
        compiler_params=pltpu.CompilerParams(needs_layout_passes=False),
        out_type=jax.ShapeDtypeStruct((_N_EDGES,), jnp.float32),
        scratch_types=[
            pltpu.VMEM((_N_NODES,), jnp.int32),
            pltpu.VMEM((_N_STARTS,), jnp.int32),
            pltpu.VMEM((_N_GRAPHS,), jnp.int32),
            pltpu.VMEM((_N_GRAPHS,), jnp.int32),
            pltpu.VMEM((_N_GRAPHS,), jnp.int32),
            pltpu.VMEM((_CHUNK,), jnp.int32),
            pltpu.VMEM((_CHUNK,), jnp.int32),
            pltpu.VMEM((_CHUNK,), jnp.int32),
            pltpu.VMEM((_CHUNK,), jnp.float32),
            pltpu.VMEM((_CHUNK,), jnp.float32),
        ],
    )
    return run(edge_scores, edge_index[0], edge_index[1], edge_batch,
               node_global_ids, start_node_locals, current_tail, prev_tail,
               step_counts)


# speedup vs baseline: 268.7316x; 1.0628x over previous
import jax
import jax.numpy as jnp
from jax import lax
from jax.experimental import pallas as pl
from jax.experimental.pallas import tpu as pltpu
from jax.experimental.pallas import tpu_sc as plsc

_N_NODES = 50000
_N_EDGES = 1600000
_N_GRAPHS = 16
_N_STARTS = 64
_LANES = 16
_NUM_TILES = 32
_EDGES_PER_TILE = _N_EDGES // _NUM_TILES
_CHUNK = 2000
_CHUNKS_PER_TILE = _EDGES_PER_TILE // _CHUNK
_VECS_PER_CHUNK = _CHUNK // _LANES
_START_BIT = 1 << 30
_GID_MASK = _START_BIT - 1


def _body(scores, heads, tails, batch, gid, starts, ct, pt, stc, out,
          table_v, starts_v, ct_v, pt_v, st_v,
          h_v, t_v, b_v, sc_v, o_v):
    wid = lax.axis_index("s") * 2 + lax.axis_index("c")
    tile_base = wid * _EDGES_PER_TILE

    pltpu.sync_copy(gid, table_v)
    pltpu.sync_copy(starts, starts_v)
    pltpu.sync_copy(ct, ct_v)
    pltpu.sync_copy(pt, pt_v)
    pltpu.sync_copy(stc, st_v)

    for j in range(_N_STARTS // _LANES):
        sidx = starts_v[pl.ds(j * _LANES, _LANES)]
        cur = plsc.load_gather(table_v, [sidx])
        plsc.store_scatter(table_v, [sidx], cur | _START_BIT)

    ctv0 = ct_v[pl.ds(0, _LANES)]
    sv0 = st_v[pl.ds(0, _LANES)]
    is0 = sv0 == 0
    ct_v[pl.ds(0, _LANES)] = jnp.where(is0, _START_BIT | _GID_MASK, ctv0)

    def chunk_body(c, carry):
        base = tile_base + c * _CHUNK
        pltpu.sync_copy(scores.at[pl.ds(base, _CHUNK)], sc_v)
        pltpu.sync_copy(heads.at[pl.ds(base, _CHUNK)], h_v)
        pltpu.sync_copy(tails.at[pl.ds(base, _CHUNK)], t_v)
        pltpu.sync_copy(batch.at[pl.ds(base, _CHUNK)], b_v)

        @plsc.parallel_loop(0, _CHUNK, _LANES, unroll=5)
        def vec_body(off):
            h = h_v[pl.ds(off, _LANES)]
            t = t_v[pl.ds(off, _LANES)]
            b = b_v[pl.ds(off, _LANES)]
            sc = sc_v[pl.ds(off, _LANES)]
            hv = plsc.load_gather(table_v, [h])
            tv = plsc.load_gather(table_v, [t])
            ctv = plsc.load_gather(ct_v, [b])
            ptv = plsc.load_gather(pt_v, [b])
            cm = ctv & _GID_MASK
            hm = hv & _GID_MASK
            tm = tv & _GID_MASK
            hc = hm == cm
            tc = tm == cm
            hp = hm == ptv
            tp = tm == ptv
            valid = (hc | tc) & jnp.logical_not((hc & tp) | (tc & hp))
            is_start = ((hv | tv) & _START_BIT) != 0
            mask = jnp.where(ctv >= _START_BIT, is_start, valid)
            o_v[pl.ds(off, _LANES)] = jnp.where(mask, sc, jnp.float32(-1e9))
        pltpu.sync_copy(o_v, out.at[pl.ds(base, _CHUNK)])
        return carry

    lax.fori_loop(0, _CHUNKS_PER_TILE, chunk_body, 0)


def kernel(edge_scores, edge_index, edge_batch, node_global_ids,
           start_node_locals, current_tail, prev_tail, step_counts):
    mesh = plsc.VectorSubcoreMesh(core_axis_name="c", subcore_axis_name="s")
    run = pl.kernel(
        _body,
        mesh=mesh,
        compiler_params=pltpu.CompilerParams(needs_layout_passes=False),
        out_type=jax.ShapeDtypeStruct((_N_EDGES,), jnp.float32),
        scratch_types=[
            pltpu.VMEM((_N_NODES,), jnp.int32),
            pltpu.VMEM((_N_STARTS,), jnp.int32),
            pltpu.VMEM((_N_GRAPHS,), jnp.int32),
            pltpu.VMEM((_N_GRAPHS,), jnp.int32),
            pltpu.VMEM((_N_GRAPHS,), jnp.int32),
            pltpu.VMEM((_CHUNK,), jnp.int32),
            pltpu.VMEM((_CHUNK,), jnp.int32),
            pltpu.VMEM((_CHUNK,), jnp.int32),
            pltpu.VMEM((_CHUNK,), jnp.float32),
            pltpu.VMEM((_CHUNK,), jnp.float32),
        ],
    )
    return run(edge_scores, edge_index[0], edge_index[1], edge_batch,
               node_global_ids, start_node_locals, current_tail, prev_tail,
               step_counts)


# speedup vs baseline: 388.8398x; 1.4469x over previous
import jax
import jax.numpy as jnp
from jax import lax
from jax.experimental import pallas as pl
from jax.experimental.pallas import tpu as pltpu
from jax.experimental.pallas import tpu_sc as plsc

_N_NODES = 50000
_N_EDGES = 1600000
_N_GRAPHS = 16
_N_STARTS = 64
_LANES = 16
_NUM_TILES = 32
_EDGES_PER_TILE = _N_EDGES // _NUM_TILES
_CHUNK = 10000
_CHUNKS_PER_TILE = _EDGES_PER_TILE // _CHUNK
_VECS_PER_CHUNK = _CHUNK // _LANES
_START_BIT = 1 << 30
_GID_MASK = _START_BIT - 1


def _body(scores, heads, tails, batch, gid, starts, ct, pt, stc, out,
          table_v, starts_v, ct_v, pt_v, st_v,
          h_v, t_v, b_v, sc_v, o_v, sem):
    wid = lax.axis_index("s") * 2 + lax.axis_index("c")
    tile_base = wid * _EDGES_PER_TILE

    pltpu.sync_copy(gid, table_v)
    pltpu.sync_copy(starts, starts_v)
    pltpu.sync_copy(ct, ct_v)
    pltpu.sync_copy(pt, pt_v)
    pltpu.sync_copy(stc, st_v)

    for j in range(_N_STARTS // _LANES):
        sidx = starts_v[pl.ds(j * _LANES, _LANES)]
        cur = plsc.load_gather(table_v, [sidx])
        plsc.store_scatter(table_v, [sidx], cur | _START_BIT)

    ctv0 = ct_v[pl.ds(0, _LANES)]
    sv0 = st_v[pl.ds(0, _LANES)]
    is0 = sv0 == 0
    ct_v[pl.ds(0, _LANES)] = jnp.where(is0, _START_BIT | _GID_MASK, ctv0)

    def chunk_body(c, carry):
        base = tile_base + c * _CHUNK
        c0 = pltpu.async_copy(scores.at[pl.ds(base, _CHUNK)], sc_v, sem)
        c1 = pltpu.async_copy(heads.at[pl.ds(base, _CHUNK)], h_v, sem)
        c2 = pltpu.async_copy(tails.at[pl.ds(base, _CHUNK)], t_v, sem)
        c3 = pltpu.async_copy(batch.at[pl.ds(base, _CHUNK)], b_v, sem)
        c0.wait()
        c1.wait()
        c2.wait()
        c3.wait()

        @plsc.parallel_loop(0, _CHUNK, _LANES, unroll=5)
        def vec_body(off):
            h = h_v[pl.ds(off, _LANES)]
            t = t_v[pl.ds(off, _LANES)]
            b = b_v[pl.ds(off, _LANES)]
            sc = sc_v[pl.ds(off, _LANES)]
            hv = plsc.load_gather(table_v, [h])
            tv = plsc.load_gather(table_v, [t])
            ctv = plsc.load_gather(ct_v, [b])
            ptv = plsc.load_gather(pt_v, [b])
            cm = ctv & _GID_MASK
            hm = hv & _GID_MASK
            tm = tv & _GID_MASK
            hc = hm == cm
            tc = tm == cm
            hp = hm == ptv
            tp = tm == ptv
            valid = (hc | tc) & jnp.logical_not((hc & tp) | (tc & hp))
            is_start = ((hv | tv) & _START_BIT) != 0
            mask = jnp.where(ctv >= _START_BIT, is_start, valid)
            o_v[pl.ds(off, _LANES)] = jnp.where(mask, sc, jnp.float32(-1e9))
        pltpu.sync_copy(o_v, out.at[pl.ds(base, _CHUNK)])
        return carry

    lax.fori_loop(0, _CHUNKS_PER_TILE, chunk_body, 0)


def kernel(edge_scores, edge_index, edge_batch, node_global_ids,
           start_node_locals, current_tail, prev_tail, step_counts):
    mesh = plsc.VectorSubcoreMesh(core_axis_name="c", subcore_axis_name="s")
    run = pl.kernel(
        _body,
        mesh=mesh,
        compiler_params=pltpu.CompilerParams(needs_layout_passes=False),
        out_type=jax.ShapeDtypeStruct((_N_EDGES,), jnp.float32),
        scratch_types=[
            pltpu.VMEM((_N_NODES,), jnp.int32),
            pltpu.VMEM((_N_STARTS,), jnp.int32),
            pltpu.VMEM((_N_GRAPHS,), jnp.int32),
            pltpu.VMEM((_N_GRAPHS,), jnp.int32),
            pltpu.VMEM((_N_GRAPHS,), jnp.int32),
            pltpu.VMEM((_CHUNK,), jnp.int32),
            pltpu.VMEM((_CHUNK,), jnp.int32),
            pltpu.VMEM((_CHUNK,), jnp.int32),
            pltpu.VMEM((_CHUNK,), jnp.float32),
            pltpu.VMEM((_CHUNK,), jnp.float32),
            pltpu.SemaphoreType.DMA,
        ],
    )
    return run(edge_scores, edge_index[0], edge_index[1], edge_batch,
               node_global_ids, start_node_locals, current_tail, prev_tail,
               step_counts)


# speedup vs baseline: 746.2549x; 1.9192x over previous
import jax
import jax.numpy as jnp
from jax import lax
from jax.experimental import pallas as pl
from jax.experimental.pallas import tpu as pltpu
from jax.experimental.pallas import tpu_sc as plsc

_N_NODES = 50000
_N_EDGES = 1600000
_N_GRAPHS = 16
_N_STARTS = 64
_LANES = 16
_NUM_TILES = 32
_CHUNK = 6400
_N_CHUNKS = _N_EDGES // _CHUNK
_ROUNDS = -(-_N_CHUNKS // _NUM_TILES)
_START_BIT = 1 << 30
_GID_MASK = _START_BIT - 1


def _body(scores, ei, batch, gid, starts, ct, pt, stc, out,
          table_v, starts_v, ct_v, pt_v, st_v,
          ei_v, b_v, sc_v, o_v, sem):
    wid = lax.axis_index("s") * 2 + lax.axis_index("c")

    pltpu.sync_copy(gid, table_v)
    pltpu.sync_copy(starts, starts_v)
    pltpu.sync_copy(ct, ct_v)
    pltpu.sync_copy(pt, pt_v)
    pltpu.sync_copy(stc, st_v)

    for j in range(_N_STARTS // _LANES):
        sidx = starts_v[pl.ds(j * _LANES, _LANES)]
        cur = plsc.load_gather(table_v, [sidx])
        plsc.store_scatter(table_v, [sidx], cur | _START_BIT)

    ctv0 = ct_v[pl.ds(0, _LANES)]
    sv0 = st_v[pl.ds(0, _LANES)]
    is0 = sv0 == 0
    ct_v[pl.ds(0, _LANES)] = jnp.where(is0, _START_BIT | _GID_MASK, ctv0)

    def chunk_body(c, carry):
        chunk_id = wid + c * _NUM_TILES

        @pl.when(chunk_id < _N_CHUNKS)
        def _():
            base = pl.multiple_of(chunk_id * _CHUNK, _CHUNK)
            c0 = pltpu.async_copy(scores.at[pl.ds(base, _CHUNK)], sc_v, sem)
            c1 = pltpu.async_copy(ei.at[:, pl.ds(base, _CHUNK)], ei_v, sem)
            c2 = pltpu.async_copy(batch.at[pl.ds(base, _CHUNK)], b_v, sem)
            c0.wait()
            c1.wait()
            c2.wait()

            @plsc.parallel_loop(0, _CHUNK, _LANES, unroll=5)
            def vec_body(off):
                h = ei_v[0, pl.ds(off, _LANES)]
                t = ei_v[1, pl.ds(off, _LANES)]
                b = b_v[pl.ds(off, _LANES)]
                sc = sc_v[pl.ds(off, _LANES)]
                hv = plsc.load_gather(table_v, [h])
                tv = plsc.load_gather(table_v, [t])
                ctv = plsc.load_gather(ct_v, [b])
                ptv = plsc.load_gather(pt_v, [b])
                cm = ctv & _GID_MASK
                hm = hv & _GID_MASK
                tm = tv & _GID_MASK
                hc = hm == cm
                tc = tm == cm
                hp = hm == ptv
                tp = tm == ptv
                valid = (hc | tc) & jnp.logical_not((hc & tp) | (tc & hp))
                is_start = ((hv | tv) & _START_BIT) != 0
                mask = jnp.where(ctv >= _START_BIT, is_start, valid)
                o_v[pl.ds(off, _LANES)] = jnp.where(mask, sc, jnp.float32(-1e9))
            pltpu.sync_copy(o_v, out.at[pl.ds(base, _CHUNK)])
        return carry

    lax.fori_loop(0, _ROUNDS, chunk_body, 0)


def kernel(edge_scores, edge_index, edge_batch, node_global_ids,
           start_node_locals, current_tail, prev_tail, step_counts):
    mesh = plsc.VectorSubcoreMesh(core_axis_name="c", subcore_axis_name="s")
    run = pl.kernel(
        _body,
        mesh=mesh,
        compiler_params=pltpu.CompilerParams(needs_layout_passes=False),
        out_type=jax.ShapeDtypeStruct((_N_EDGES,), jnp.float32),
        scratch_types=[
            pltpu.VMEM((_N_NODES,), jnp.int32),
            pltpu.VMEM((_N_STARTS,), jnp.int32),
            pltpu.VMEM((_N_GRAPHS,), jnp.int32),
            pltpu.VMEM((_N_GRAPHS,), jnp.int32),
            pltpu.VMEM((_N_GRAPHS,), jnp.int32),
            pltpu.VMEM((2, _CHUNK), jnp.int32),
            pltpu.VMEM((_CHUNK,), jnp.int32),
            pltpu.VMEM((_CHUNK,), jnp.float32),
            pltpu.VMEM((_CHUNK,), jnp.float32),
            pltpu.SemaphoreType.DMA,
        ],
    )
    return run(edge_scores, edge_index, edge_batch, node_global_ids,
               start_node_locals, current_tail, prev_tail, step_counts)
